# Initial kernel scaffold; baseline (speedup 1.0000x reference)
import jax
import jax.numpy as jnp
from jax.experimental import pallas as pl


def kernel(r_mol, r_molecule_index, a_mol, a_molecule_index, p_mol, p_molecule_index, device, r_W1, r_b1, r_W2, r_b2, a_W1, a_b1, a_W2, a_b2, p_W1, p_b1, p_W2, p_b2):
    raise NotImplementedError("write your pallas kernel here")


# speedup vs baseline: 2.5086x; 2.5086x over previous
import jax
import jax.numpy as jnp
from jax.experimental import pallas as pl
from jax.experimental.pallas import tpu as pltpu

IN_DIM = 128
OUT_DIM = 128
HID = IN_DIM // 4
N = 50000
S = 2048

TILE = 512
GRID = (N + TILE - 1) // TILE
N_PAD = GRID * TILE
SCALE = 1.0 / (3.0 * OUT_DIM)


def _stream_contrib(x, idx, row_valid, W1, b1, W2, b2):
    h = jax.lax.dot_general(x, W1, (((1,), (1,)), ((), ())),
                            preferred_element_type=jnp.float32)
    h = jax.nn.relu(h + b1)
    u = jax.lax.dot_general(h, W2, (((1,), (1,)), ((), ())),
                            preferred_element_type=jnp.float32)
    u = jax.nn.sigmoid(u + b2)
    c = jnp.sum(u, axis=1, keepdims=True) * SCALE
    z = jnp.where(row_valid, c * x, 0.0)
    onehot = (idx[:, None] ==
              jax.lax.broadcasted_iota(jnp.int32, (TILE, S), 1)
              ).astype(jnp.float32)
    return jax.lax.dot_general(onehot, z, (((0,), (0,)), ((), ())),
                               preferred_element_type=jnp.float32)


def _body(ri, ai, pi, rx, ax, px, rW1, rb1, rW2, rb2, aW1, ab1, aW2, ab2,
          pW1, pb1, pW2, pb2, out_ref):
    i = pl.program_id(0)
    row = i * TILE + jax.lax.broadcasted_iota(jnp.int32, (TILE, 1), 0)
    row_valid = row < N
    acc = _stream_contrib(rx[...], ri[0, 0], row_valid, rW1[...], rb1[...],
                          rW2[...], rb2[...])
    acc += _stream_contrib(ax[...], ai[0, 0], row_valid, aW1[...], ab1[...],
                           aW2[...], ab2[...])
    acc += _stream_contrib(px[...], pi[0, 0], row_valid, pW1[...], pb1[...],
                           pW2[...], pb2[...])

    @pl.when(i == 0)
    def _():
        out_ref[...] = jnp.zeros_like(out_ref)

    out_ref[...] += acc


def kernel(r_mol, r_molecule_index, a_mol, a_molecule_index, p_mol,
           p_molecule_index, device, r_W1, r_b1, r_W2, r_b2, a_W1, a_b1,
           a_W2, a_b2, p_W1, p_b1, p_W2, p_b2):
    del device

    def pad_idx(idx):
        idx = jnp.concatenate([idx.astype(jnp.int32),
                               jnp.zeros((N_PAD - N,), jnp.int32)])
        return idx.reshape(GRID, 1, TILE)

    ri = pad_idx(r_molecule_index)
    ai = pad_idx(a_molecule_index)
    pi = pad_idx(p_molecule_index)

    row_block = pl.BlockSpec((TILE, IN_DIM), lambda i: (i, 0))
    idx_block = pl.BlockSpec((1, 1, TILE), lambda i: (i, 0, 0))
    w1_block = pl.BlockSpec((HID, IN_DIM), lambda i: (0, 0))
    b1_block = pl.BlockSpec((1, HID), lambda i: (0, 0))
    w2_block = pl.BlockSpec((OUT_DIM, HID), lambda i: (0, 0))
    b2_block = pl.BlockSpec((1, OUT_DIM), lambda i: (0, 0))

    out = pl.pallas_call(
        _body,
        grid=(GRID,),
        in_specs=[idx_block, idx_block, idx_block,
                  row_block, row_block, row_block,
                  w1_block, b1_block, w2_block, b2_block,
                  w1_block, b1_block, w2_block, b2_block,
                  w1_block, b1_block, w2_block, b2_block],
        out_specs=pl.BlockSpec((S, OUT_DIM), lambda i: (0, 0)),
        out_shape=jax.ShapeDtypeStruct((S, OUT_DIM), jnp.float32),
    )(ri, ai, pi, r_mol, a_mol, p_mol,
      r_W1, r_b1.reshape(1, HID), r_W2, r_b2.reshape(1, OUT_DIM),
      a_W1, a_b1.reshape(1, HID), a_W2, a_b2.reshape(1, OUT_DIM),
      p_W1, p_b1.reshape(1, HID), p_W2, p_b2.reshape(1, OUT_DIM))
    return out


# speedup vs baseline: 3.6175x; 1.4421x over previous
import functools

import jax
import jax.numpy as jnp
from jax import lax
from jax.experimental import pallas as pl
from jax.experimental.pallas import tpu as pltpu
from jax.experimental.pallas import tpu_sc as plsc

IN_DIM = 128
OUT_DIM = 128
HID = IN_DIM // 4
N = 50000
S = 2048

TILE = 512
GRID = (N + TILE - 1) // TILE
N_PAD = GRID * TILE
SCALE = 1.0 / (3.0 * OUT_DIM)

NC = 2
NS = 16
NW = NC * NS
RPW = N_PAD // NW
CH = 112
NCHUNK = RPW // CH
ROWS_PER_TILE = S // NS


def _z_stream(x, row_valid, W1, b1, W2, b2):
    h = lax.dot_general(x, W1, (((1,), (1,)), ((), ())),
                        preferred_element_type=jnp.float32)
    h = jax.nn.relu(h + b1)
    u = lax.dot_general(h, W2, (((1,), (1,)), ((), ())),
                        preferred_element_type=jnp.float32)
    u = jax.nn.sigmoid(u + b2)
    c = jnp.sum(u, axis=1, keepdims=True) * SCALE
    return jnp.where(row_valid, c * x, 0.0)


def _z_body(rx, ax, px, rW1, rb1, rW2, rb2, aW1, ab1, aW2, ab2,
            pW1, pb1, pW2, pb2, zr, za, zp):
    i = pl.program_id(0)
    row = i * TILE + lax.broadcasted_iota(jnp.int32, (TILE, 1), 0)
    row_valid = row < N
    zr[...] = _z_stream(rx[...], row_valid, rW1[...], rb1[...], rW2[...], rb2[...])
    za[...] = _z_stream(ax[...], row_valid, aW1[...], ab1[...], aW2[...], ab2[...])
    zp[...] = _z_stream(px[...], row_valid, pW1[...], pb1[...], pW2[...], pb2[...])


def _sc_body(zr, ir, za, ia, zp, ip, out, acc, zbuf, ibuf, tbuf):
    cid = lax.axis_index("c")
    sid = lax.axis_index("s")
    wid = cid * NS + sid
    base = wid * RPW

    def _zero(i, _):
        r = i // 8
        col = (i % 8) * 16
        tbuf[r, pl.ds(col, 16)] = jnp.zeros((16,), jnp.float32)
        return 0

    lax.fori_loop(0, ROWS_PER_TILE * 8, _zero, 0)
    pltpu.sync_copy(tbuf, acc.at[pl.ds(sid * ROWS_PER_TILE, ROWS_PER_TILE)])
    plsc.subcore_barrier()

    def _chunk(k, _):
        off = base + k * CH
        for z_hbm, i_hbm in ((zr, ir), (za, ia), (zp, ip)):
            pltpu.sync_copy(i_hbm.at[pl.ds(off, CH)], ibuf)
            pltpu.sync_copy(z_hbm.at[pl.ds(off, CH)], zbuf)
            pltpu.sync_copy(zbuf, acc.at[ibuf], add=True)
        return 0

    lax.fori_loop(0, NCHUNK, _chunk, 0)
    plsc.subcore_barrier()

    r0 = sid * ROWS_PER_TILE
    pltpu.sync_copy(acc.at[pl.ds(r0, ROWS_PER_TILE)],
                    out.at[cid, pl.ds(r0, ROWS_PER_TILE)])


def _add_body(p_ref, out_ref):
    out_ref[...] = p_ref[0] + p_ref[1]


def kernel(r_mol, r_molecule_index, a_mol, a_molecule_index, p_mol,
           p_molecule_index, device, r_W1, r_b1, r_W2, r_b2, a_W1, a_b1,
           a_W2, a_b2, p_W1, p_b1, p_W2, p_b2):
    del device

    row_block = pl.BlockSpec((TILE, IN_DIM), lambda i: (i, 0))
    w1_block = pl.BlockSpec((HID, IN_DIM), lambda i: (0, 0))
    b1_block = pl.BlockSpec((1, HID), lambda i: (0, 0))
    w2_block = pl.BlockSpec((OUT_DIM, HID), lambda i: (0, 0))
    b2_block = pl.BlockSpec((1, OUT_DIM), lambda i: (0, 0))

    zs = pl.pallas_call(
        _z_body,
        grid=(GRID,),
        in_specs=[row_block, row_block, row_block,
                  w1_block, b1_block, w2_block, b2_block,
                  w1_block, b1_block, w2_block, b2_block,
                  w1_block, b1_block, w2_block, b2_block],
        out_specs=[row_block, row_block, row_block],
        out_shape=[jax.ShapeDtypeStruct((N_PAD, IN_DIM), jnp.float32)] * 3,
    )(r_mol, a_mol, p_mol,
      r_W1, r_b1.reshape(1, HID), r_W2, r_b2.reshape(1, OUT_DIM),
      a_W1, a_b1.reshape(1, HID), a_W2, a_b2.reshape(1, OUT_DIM),
      p_W1, p_b1.reshape(1, HID), p_W2, p_b2.reshape(1, OUT_DIM))
    zr, za, zp = zs

    pad = jnp.zeros((N_PAD - N,), jnp.int32)
    ir = jnp.concatenate([r_molecule_index.astype(jnp.int32), pad])
    ia = jnp.concatenate([a_molecule_index.astype(jnp.int32), pad])
    ip = jnp.concatenate([p_molecule_index.astype(jnp.int32), pad])

    sc = functools.partial(
        pl.kernel,
        out_type=jax.ShapeDtypeStruct((NC, S, OUT_DIM), jnp.float32),
        mesh=plsc.VectorSubcoreMesh(core_axis_name="c", subcore_axis_name="s"),
        scratch_types=[
            pltpu.VMEM_SHARED((S, OUT_DIM), jnp.float32),
            pltpu.VMEM((CH, OUT_DIM), jnp.float32),
            pltpu.VMEM((CH,), jnp.int32),
            pltpu.VMEM((ROWS_PER_TILE, OUT_DIM), jnp.float32),
        ],
    )
    partials = sc(_sc_body)(zr, ir, za, ia, zp, ip)

    return pl.pallas_call(
        _add_body,
        in_specs=[pl.BlockSpec((NC, S, OUT_DIM), lambda: (0, 0, 0))],
        out_specs=pl.BlockSpec((S, OUT_DIM), lambda: (0, 0)),
        out_shape=jax.ShapeDtypeStruct((S, OUT_DIM), jnp.float32),
    )(partials)


# speedup vs baseline: 4.6246x; 1.2784x over previous
import functools

import jax
import jax.numpy as jnp
from jax import lax
from jax.experimental import pallas as pl
from jax.experimental.pallas import tpu as pltpu
from jax.experimental.pallas import tpu_sc as plsc

IN_DIM = 128
OUT_DIM = 128
HID = IN_DIM // 4
N = 50000
S = 2048

TILE = 512
N_PAD = 50176
GRID = N_PAD // TILE
SCALE = 1.0 / (3.0 * OUT_DIM)

NC = 2
NS = 16
NW = NC * NS
RPW = N_PAD // NW
CH = 112
NCHUNK = RPW // CH
NQ = 3 * NCHUNK
DEPTH = 3
ROWS_PER_TILE = S // NS


def _z_stream(x, row_valid, W1, b1, W2, b2):
    h = lax.dot_general(x, W1, (((1,), (1,)), ((), ())),
                        preferred_element_type=jnp.float32)
    h = jax.nn.relu(h + b1)
    u = lax.dot_general(h, W2, (((1,), (1,)), ((), ())),
                        preferred_element_type=jnp.float32)
    u = jax.nn.sigmoid(u + b2)
    c = jnp.sum(u, axis=1, keepdims=True) * SCALE
    return jnp.where(row_valid, c * x, 0.0)


def _z_body(rx, ax, px, rW1, rb1, rW2, rb2, aW1, ab1, aW2, ab2,
            pW1, pb1, pW2, pb2, zr, za, zp):
    i = pl.program_id(0)
    row = i * TILE + lax.broadcasted_iota(jnp.int32, (TILE, 1), 0)
    row_valid = row < N
    zr[...] = _z_stream(rx[...], row_valid, rW1[...], rb1[...], rW2[...], rb2[...])
    za[...] = _z_stream(ax[...], row_valid, aW1[...], ab1[...], aW2[...], ab2[...])
    zp[...] = _z_stream(px[...], row_valid, pW1[...], pb1[...], pW2[...], pb2[...])


def _sc_body(zr, ir, za, ia, zp, ip, zeros, out, acc,
             zb0, zb1, zb2, ib0, ib1, ib2,
             zs0, zs1, zs2, is0, is1, is2, ss0, ss1, ss2):
    cid = lax.axis_index("c")
    sid = lax.axis_index("s")
    wid = cid * NS + sid
    r0 = sid * ROWS_PER_TILE

    zbs = (zb0, zb1, zb2)
    ibs = (ib0, ib1, ib2)
    zsems = (zs0, zs1, zs2)
    isems = (is0, is1, is2)
    ssems = (ss0, ss1, ss2)
    zsrcs = (zr, za, zp)
    isrcs = (ir, ia, ip)

    for h in range(0, ROWS_PER_TILE, 64):
        pltpu.sync_copy(zeros.at[pl.ds(r0 + h, 64)], zb0.at[pl.ds(0, 64)])
        pltpu.sync_copy(zb0.at[pl.ds(0, 64)], acc.at[pl.ds(r0 + h, 64)])
    plsc.subcore_barrier()

    def _start(q, b):
        m, k = q // NCHUNK, q % NCHUNK
        off = wid * RPW + k * CH
        zh = pltpu.async_copy(zsrcs[m].at[pl.ds(off, CH)], zbs[b], zsems[b])
        ih = pltpu.async_copy(isrcs[m].at[pl.ds(off, CH)], ibs[b], isems[b])
        return zh, ih

    gh = {0: _start(0, 0), 1: _start(1, 1)}
    sh = {}
    for q in range(NQ):
        b = q % DEPTH
        if q + 2 < NQ:
            nb = (q + 2) % DEPTH
            if nb in sh:
                sh.pop(nb).wait()
            gh[nb] = _start(q + 2, nb)
        zh, ih = gh.pop(b)
        zh.wait()
        ih.wait()
        sh[b] = pltpu.async_copy(zbs[b], acc.at[ibs[b]], ssems[b], add=True)
    for b in sorted(sh):
        sh.pop(b).wait()
    plsc.subcore_barrier()

    pltpu.sync_copy(acc.at[pl.ds(r0, ROWS_PER_TILE)],
                    out.at[cid, pl.ds(r0, ROWS_PER_TILE)])


def _add_body(p_ref, out_ref):
    out_ref[...] = p_ref[0] + p_ref[1]


def kernel(r_mol, r_molecule_index, a_mol, a_molecule_index, p_mol,
           p_molecule_index, device, r_W1, r_b1, r_W2, r_b2, a_W1, a_b1,
           a_W2, a_b2, p_W1, p_b1, p_W2, p_b2):
    del device

    row_block = pl.BlockSpec((TILE, IN_DIM), lambda i: (i, 0))
    w1_block = pl.BlockSpec((HID, IN_DIM), lambda i: (0, 0))
    b1_block = pl.BlockSpec((1, HID), lambda i: (0, 0))
    w2_block = pl.BlockSpec((OUT_DIM, HID), lambda i: (0, 0))
    b2_block = pl.BlockSpec((1, OUT_DIM), lambda i: (0, 0))

    zs = pl.pallas_call(
        _z_body,
        grid=(GRID,),
        in_specs=[row_block, row_block, row_block,
                  w1_block, b1_block, w2_block, b2_block,
                  w1_block, b1_block, w2_block, b2_block,
                  w1_block, b1_block, w2_block, b2_block],
        out_specs=[row_block, row_block, row_block],
        out_shape=[jax.ShapeDtypeStruct((N_PAD, IN_DIM), jnp.float32)] * 3,
    )(r_mol, a_mol, p_mol,
      r_W1, r_b1.reshape(1, HID), r_W2, r_b2.reshape(1, OUT_DIM),
      a_W1, a_b1.reshape(1, HID), a_W2, a_b2.reshape(1, OUT_DIM),
      p_W1, p_b1.reshape(1, HID), p_W2, p_b2.reshape(1, OUT_DIM))
    zr, za, zp = zs

    pad = jnp.zeros((N_PAD - N,), jnp.int32)
    ir = jnp.concatenate([r_molecule_index.astype(jnp.int32), pad])
    ia = jnp.concatenate([a_molecule_index.astype(jnp.int32), pad])
    ip = jnp.concatenate([p_molecule_index.astype(jnp.int32), pad])
    zeros = jnp.zeros((S, OUT_DIM), jnp.float32)

    sc = functools.partial(
        pl.kernel,
        out_type=jax.ShapeDtypeStruct((NC, S, OUT_DIM), jnp.float32),
        mesh=plsc.VectorSubcoreMesh(core_axis_name="c", subcore_axis_name="s"),
        scratch_types=(
            [pltpu.VMEM_SHARED((S, OUT_DIM), jnp.float32)]
            + [pltpu.VMEM((CH, OUT_DIM), jnp.float32)] * DEPTH
            + [pltpu.VMEM((CH,), jnp.int32)] * DEPTH
            + [pltpu.SemaphoreType.DMA] * (3 * DEPTH)
        ),
    )
    partials = sc(_sc_body)(zr, ir, za, ia, zp, ip, zeros)

    return pl.pallas_call(
        _add_body,
        in_specs=[pl.BlockSpec((NC, S, OUT_DIM), lambda: (0, 0, 0))],
        out_specs=pl.BlockSpec((S, OUT_DIM), lambda: (0, 0)),
        out_shape=jax.ShapeDtypeStruct((S, OUT_DIM), jnp.float32),
    )(partials)


# speedup vs baseline: 5.5616x; 1.2026x over previous
import functools

import jax
import jax.numpy as jnp
from jax import lax
from jax.experimental import pallas as pl
from jax.experimental.pallas import tpu as pltpu
from jax.experimental.pallas import tpu_sc as plsc

IN_DIM = 128
OUT_DIM = 128
HID = IN_DIM // 4
N = 50000
S = 2048

TILE = 1024
N_PAD = 50176
GRID = N_PAD // TILE
SCALE = 1.0 / (3.0 * OUT_DIM)

NC = 2
NS = 16
NW = NC * NS
RPW = N_PAD // NW
CH = 112
NCHUNK = RPW // CH
NQ = 3 * NCHUNK
DEPTH = 4
PREF = DEPTH - 1
ROWS_PER_TILE = S // NS


def _z_stream(x, row_valid, W1, b1, W2, b2):
    h = lax.dot_general(x, W1, (((1,), (1,)), ((), ())),
                        preferred_element_type=jnp.float32)
    h = jax.nn.relu(h + b1)
    u = lax.dot_general(h, W2, (((1,), (1,)), ((), ())),
                        preferred_element_type=jnp.float32)
    u = jax.nn.sigmoid(u + b2)
    c = jnp.sum(u, axis=1, keepdims=True) * SCALE
    return jnp.where(row_valid, c * x, 0.0)


def _z_body(rx, ax, px, rW1, rb1, rW2, rb2, aW1, ab1, aW2, ab2,
            pW1, pb1, pW2, pb2, zr, za, zp):
    i = pl.program_id(0)
    row = i * TILE + lax.broadcasted_iota(jnp.int32, (TILE, 1), 0)
    row_valid = row < N
    zr[...] = _z_stream(rx[...], row_valid, rW1[...], rb1[...], rW2[...], rb2[...])
    za[...] = _z_stream(ax[...], row_valid, aW1[...], ab1[...], aW2[...], ab2[...])
    zp[...] = _z_stream(px[...], row_valid, pW1[...], pb1[...], pW2[...], pb2[...])


def _sc_body(zr, ir, za, ia, zp, ip, zeros, out, acc, *rest):
    cid = lax.axis_index("c")
    sid = lax.axis_index("s")
    wid = cid * NS + sid
    r0 = sid * ROWS_PER_TILE

    zbs = rest[:DEPTH]
    ibs = rest[DEPTH:2 * DEPTH]
    zsems = rest[2 * DEPTH:3 * DEPTH]
    isems = rest[3 * DEPTH:4 * DEPTH]
    ssems = rest[4 * DEPTH:5 * DEPTH]
    zb0 = zbs[0]
    zsrcs = (zr, za, zp)
    isrcs = (ir, ia, ip)

    for h in range(0, ROWS_PER_TILE, 64):
        pltpu.sync_copy(zeros.at[pl.ds(r0 + h, 64)], zb0.at[pl.ds(0, 64)])
        pltpu.sync_copy(zb0.at[pl.ds(0, 64)], acc.at[pl.ds(r0 + h, 64)])
    plsc.subcore_barrier()

    def _start(q, b):
        m, k = q // NCHUNK, q % NCHUNK
        off = wid * RPW + k * CH
        zh = pltpu.async_copy(zsrcs[m].at[pl.ds(off, CH)], zbs[b], zsems[b])
        ih = pltpu.async_copy(isrcs[m].at[pl.ds(off, CH)], ibs[b], isems[b])
        return zh, ih

    gh = {b: _start(b, b) for b in range(PREF)}
    sh = {}
    for q in range(NQ):
        b = q % DEPTH
        if q + PREF < NQ:
            nb = (q + PREF) % DEPTH
            if nb in sh:
                sh.pop(nb).wait()
            gh[nb] = _start(q + PREF, nb)
        zh, ih = gh.pop(b)
        zh.wait()
        ih.wait()
        sh[b] = pltpu.async_copy(zbs[b], acc.at[ibs[b]], ssems[b], add=True)
    for b in sorted(sh):
        sh.pop(b).wait()
    plsc.subcore_barrier()

    pltpu.sync_copy(acc.at[pl.ds(r0, ROWS_PER_TILE)],
                    out.at[cid, pl.ds(r0, ROWS_PER_TILE)])


def _add_body(p_ref, out_ref):
    out_ref[...] = p_ref[0] + p_ref[1]


def kernel(r_mol, r_molecule_index, a_mol, a_molecule_index, p_mol,
           p_molecule_index, device, r_W1, r_b1, r_W2, r_b2, a_W1, a_b1,
           a_W2, a_b2, p_W1, p_b1, p_W2, p_b2):
    del device

    row_block = pl.BlockSpec((TILE, IN_DIM), lambda i: (i, 0))
    w1_block = pl.BlockSpec((HID, IN_DIM), lambda i: (0, 0))
    b1_block = pl.BlockSpec((1, HID), lambda i: (0, 0))
    w2_block = pl.BlockSpec((OUT_DIM, HID), lambda i: (0, 0))
    b2_block = pl.BlockSpec((1, OUT_DIM), lambda i: (0, 0))

    zs = pl.pallas_call(
        _z_body,
        grid=(GRID,),
        in_specs=[row_block, row_block, row_block,
                  w1_block, b1_block, w2_block, b2_block,
                  w1_block, b1_block, w2_block, b2_block,
                  w1_block, b1_block, w2_block, b2_block],
        out_specs=[row_block, row_block, row_block],
        out_shape=[jax.ShapeDtypeStruct((N_PAD, IN_DIM), jnp.float32)] * 3,
    )(r_mol, a_mol, p_mol,
      r_W1, r_b1.reshape(1, HID), r_W2, r_b2.reshape(1, OUT_DIM),
      a_W1, a_b1.reshape(1, HID), a_W2, a_b2.reshape(1, OUT_DIM),
      p_W1, p_b1.reshape(1, HID), p_W2, p_b2.reshape(1, OUT_DIM))
    zr, za, zp = zs

    pad = jnp.zeros((N_PAD - N,), jnp.int32)
    ir = jnp.concatenate([r_molecule_index.astype(jnp.int32), pad])
    ia = jnp.concatenate([a_molecule_index.astype(jnp.int32), pad])
    ip = jnp.concatenate([p_molecule_index.astype(jnp.int32), pad])
    zeros = jnp.zeros((S, OUT_DIM), jnp.float32)

    sc = functools.partial(
        pl.kernel,
        out_type=jax.ShapeDtypeStruct((NC, S, OUT_DIM), jnp.float32),
        mesh=plsc.VectorSubcoreMesh(core_axis_name="c", subcore_axis_name="s"),
        scratch_types=(
            [pltpu.VMEM_SHARED((S, OUT_DIM), jnp.float32)]
            + [pltpu.VMEM((CH, OUT_DIM), jnp.float32)] * DEPTH
            + [pltpu.VMEM((CH,), jnp.int32)] * DEPTH
            + [pltpu.SemaphoreType.DMA] * (3 * DEPTH)
        ),
    )
    partials = sc(_sc_body)(zr, ir, za, ia, zp, ip, zeros)

    return pl.pallas_call(
        _add_body,
        in_specs=[pl.BlockSpec((NC, S, OUT_DIM), lambda: (0, 0, 0))],
        out_specs=pl.BlockSpec((S, OUT_DIM), lambda: (0, 0)),
        out_shape=jax.ShapeDtypeStruct((S, OUT_DIM), jnp.float32),
    )(partials)
